# baseline (speedup 1.0000x reference)
import jax
import jax.numpy as jnp
from jax.experimental import pallas as pl
from jax.experimental.pallas import tpu as pltpu

N, NFEAT, NHID, NCLASS = 8192, 256, 64, 16

BLKA = 256
NBLKA = N // BLKA
BLKB = 512
NBLKB = N // BLKB


def _layer1_kernel(adj_ref, x_ref, W1_ref, b1_ref, W2_ref,
                   adjb_ref, h2_ref, o1_ref, h1_ref, acc_ref):
    j = pl.program_id(0)

    @pl.when(j == 0)
    def _():
        h1_ref[...] = jnp.dot(x_ref[...], W1_ref[...],
                              preferred_element_type=jnp.float32
                              ).astype(jnp.bfloat16)

    ab = adj_ref[...].astype(jnp.bfloat16)
    adjb_ref[...] = ab
    y = jnp.dot(ab, h1_ref[...], preferred_element_type=jnp.float32)
    yr = jnp.maximum(y + b1_ref[...], 0.0)
    h2_ref[...] = jnp.dot(yr, W2_ref[...],
                          preferred_element_type=jnp.float32)
    m = jnp.max(yr, axis=0, keepdims=True)

    @pl.when(j == 0)
    def _():
        acc_ref[...] = m

    @pl.when(j != 0)
    def _():
        acc_ref[...] = jnp.maximum(acc_ref[...], m)

    @pl.when(j == NBLKA - 1)
    def _():
        o1_ref[...] = acc_ref[...]


def _layer23_kernel(adjb_ref, h2_ref, W3_ref, b2_ref, b3_ref, linW_ref,
                    linb_ref, o1_ref, out_ref,
                    hcur_ref, h3_ref, acc2_ref, acc3_ref):
    i = pl.program_id(0)
    j = jax.lax.rem(i, NBLKB)
    l = jax.lax.div(i, NBLKB)

    @pl.when(i == 0)
    def _():
        hcur_ref[...] = h2_ref[...].astype(jnp.bfloat16)

    @pl.when(i == NBLKB)
    def _():
        hcur_ref[...] = h3_ref[...]

    y = jnp.dot(adjb_ref[...], hcur_ref[...],
                preferred_element_type=jnp.float32)
    y = y + jnp.where(l == 0, b2_ref[...], b3_ref[...])
    yr = jnp.maximum(y, 0.0)

    @pl.when(l == 0)
    def _():
        h3_ref[pl.ds(j * BLKB, BLKB), :] = jnp.dot(
            yr, W3_ref[...],
            preferred_element_type=jnp.float32).astype(jnp.bfloat16)

    m = jnp.max(jnp.where(l == 0, yr, y), axis=0, keepdims=True)

    def upd(acc_ref):
        @pl.when(j == 0)
        def _():
            acc_ref[...] = m

        @pl.when(j != 0)
        def _():
            acc_ref[...] = jnp.maximum(acc_ref[...], m)

    @pl.when(l == 0)
    def _():
        upd(acc2_ref)

    @pl.when(l == 1)
    def _():
        upd(acc3_ref)

    @pl.when(i == 2 * NBLKB - 1)
    def _():
        logits = (jnp.sum(linW_ref[:, 0:NHID] * o1_ref[...], axis=1)
                  + jnp.sum(linW_ref[:, NHID:2 * NHID] * acc2_ref[...], axis=1)
                  + jnp.sum(linW_ref[:, 2 * NHID:] * acc3_ref[...], axis=1)
                  + linb_ref[0, :])
        z = logits - jnp.max(logits)
        out_ref[0, :] = z - jnp.log(jnp.sum(jnp.exp(z)))


def kernel(x, adj, W1, b1, W2, b2, W3, b3, linW, linb):
    full = lambda shape: pl.BlockSpec(shape, lambda i: (0, 0))

    adj_bf16, h2, o1 = pl.pallas_call(
        _layer1_kernel,
        grid=(NBLKA,),
        in_specs=[
            pl.BlockSpec((BLKA, N), lambda j: (j, 0)),
            full((N, NFEAT)),
            full((NFEAT, NHID)),
            full((1, NHID)),
            full((NHID, NHID)),
        ],
        out_specs=[
            pl.BlockSpec((BLKA, N), lambda j: (j, 0)),
            pl.BlockSpec((BLKA, NHID), lambda j: (j, 0)),
            pl.BlockSpec((1, NHID), lambda j: (0, 0)),
        ],
        out_shape=[
            jax.ShapeDtypeStruct((N, N), jnp.bfloat16),
            jax.ShapeDtypeStruct((N, NHID), jnp.float32),
            jax.ShapeDtypeStruct((1, NHID), jnp.float32),
        ],
        scratch_shapes=[
            pltpu.VMEM((N, NHID), jnp.bfloat16),
            pltpu.VMEM((1, NHID), jnp.float32),
        ],
        compiler_params=pltpu.CompilerParams(
            dimension_semantics=("arbitrary",)),
    )(adj, x, W1, b1.reshape(1, -1), W2)

    out = pl.pallas_call(
        _layer23_kernel,
        grid=(2 * NBLKB,),
        in_specs=[
            pl.BlockSpec((BLKB, N), lambda i: (jax.lax.rem(i, NBLKB), 0)),
            full((N, NHID)),
            full((NHID, NHID)),
            full((1, NHID)),
            full((1, NHID)),
            full((NCLASS, 3 * NHID)),
            full((1, NCLASS)),
            full((1, NHID)),
        ],
        out_specs=pl.BlockSpec((1, NCLASS), lambda i: (0, 0)),
        out_shape=jax.ShapeDtypeStruct((1, NCLASS), jnp.float32),
        scratch_shapes=[
            pltpu.VMEM((N, NHID), jnp.bfloat16),
            pltpu.VMEM((N, NHID), jnp.bfloat16),
            pltpu.VMEM((1, NHID), jnp.float32),
            pltpu.VMEM((1, NHID), jnp.float32),
        ],
        compiler_params=pltpu.CompilerParams(
            dimension_semantics=("arbitrary",)),
    )(adj_bf16, h2, W3, b2.reshape(1, -1), b3.reshape(1, -1), linW,
      linb.reshape(1, -1), o1)
    return out.reshape(NCLASS)


# speedup vs baseline: 1.0510x; 1.0510x over previous
import jax
import jax.numpy as jnp
from jax.experimental import pallas as pl
from jax.experimental.pallas import tpu as pltpu

N, NFEAT, NHID, NCLASS = 8192, 256, 64, 16

BLKA = 256
NBLKA = N // BLKA
BLKB = 1024
NBLKB = N // BLKB


def _layer1_kernel(adj_ref, x_ref, W1_ref, b1_ref,
                   adjb_ref, x1_ref, o1_ref, h1_ref, acc_ref):
    j = pl.program_id(0)

    @pl.when(j == 0)
    def _():
        h1_ref[...] = jnp.dot(x_ref[...], W1_ref[...],
                              preferred_element_type=jnp.float32
                              ).astype(jnp.bfloat16)

    ab = adj_ref[...].astype(jnp.bfloat16)
    adjb_ref[...] = ab
    y = jnp.dot(ab, h1_ref[...], preferred_element_type=jnp.float32)
    yr = jnp.maximum(y + b1_ref[...], 0.0)
    x1_ref[...] = yr.astype(jnp.bfloat16)

    @pl.when(j == 0)
    def _():
        acc_ref[...] = yr

    @pl.when(j != 0)
    def _():
        acc_ref[...] = jnp.maximum(acc_ref[...], yr)

    @pl.when(j == NBLKA - 1)
    def _():
        o1_ref[...] = jnp.max(acc_ref[...], axis=0, keepdims=True)


def _layer23_kernel(adjb_ref, x1_ref, W2_ref, W3_ref, b2_ref, b3_ref,
                    linW_ref, linb_ref, o1_ref, out_ref,
                    hcur_ref, h3_ref, acc2_ref, acc3_ref):
    i = pl.program_id(0)
    j = jax.lax.rem(i, NBLKB)
    l = jax.lax.div(i, NBLKB)

    @pl.when(i == 0)
    def _():
        hcur_ref[...] = jnp.dot(x1_ref[...], W2_ref[...],
                                preferred_element_type=jnp.float32
                                ).astype(jnp.bfloat16)

    @pl.when(i == NBLKB)
    def _():
        hcur_ref[...] = h3_ref[...]

    y = jnp.dot(adjb_ref[...], hcur_ref[...],
                preferred_element_type=jnp.float32)
    y = y + jnp.where(l == 0, b2_ref[...], b3_ref[...])
    yr = jnp.maximum(y, 0.0)

    @pl.when(l == 0)
    def _():
        h3_ref[pl.ds(j * BLKB, BLKB), :] = jnp.dot(
            yr, W3_ref[...],
            preferred_element_type=jnp.float32).astype(jnp.bfloat16)

        @pl.when(j == 0)
        def _():
            acc2_ref[...] = yr

        @pl.when(j != 0)
        def _():
            acc2_ref[...] = jnp.maximum(acc2_ref[...], yr)

    @pl.when(l == 1)
    def _():
        @pl.when(j == 0)
        def _():
            acc3_ref[...] = y

        @pl.when(j != 0)
        def _():
            acc3_ref[...] = jnp.maximum(acc3_ref[...], y)

    @pl.when(i == 2 * NBLKB - 1)
    def _():
        o2 = jnp.max(acc2_ref[...], axis=0, keepdims=True)
        o3 = jnp.max(acc3_ref[...], axis=0, keepdims=True)
        logits = (jnp.sum(linW_ref[:, 0:NHID] * o1_ref[...], axis=1)
                  + jnp.sum(linW_ref[:, NHID:2 * NHID] * o2, axis=1)
                  + jnp.sum(linW_ref[:, 2 * NHID:] * o3, axis=1)
                  + linb_ref[0, :])
        z = logits - jnp.max(logits)
        out_ref[0, :] = z - jnp.log(jnp.sum(jnp.exp(z)))


def kernel(x, adj, W1, b1, W2, b2, W3, b3, linW, linb):
    full = lambda shape: pl.BlockSpec(shape, lambda i: (0, 0))

    adj_bf16, x1, o1 = pl.pallas_call(
        _layer1_kernel,
        grid=(NBLKA,),
        in_specs=[
            pl.BlockSpec((BLKA, N), lambda j: (j, 0)),
            full((N, NFEAT)),
            full((NFEAT, NHID)),
            full((1, NHID)),
        ],
        out_specs=[
            pl.BlockSpec((BLKA, N), lambda j: (j, 0)),
            pl.BlockSpec((BLKA, NHID), lambda j: (j, 0)),
            pl.BlockSpec((1, NHID), lambda j: (0, 0)),
        ],
        out_shape=[
            jax.ShapeDtypeStruct((N, N), jnp.bfloat16),
            jax.ShapeDtypeStruct((N, NHID), jnp.bfloat16),
            jax.ShapeDtypeStruct((1, NHID), jnp.float32),
        ],
        scratch_shapes=[
            pltpu.VMEM((N, NHID), jnp.bfloat16),
            pltpu.VMEM((BLKA, NHID), jnp.float32),
        ],
        compiler_params=pltpu.CompilerParams(
            dimension_semantics=("arbitrary",)),
    )(adj, x, W1, b1.reshape(1, -1))

    out = pl.pallas_call(
        _layer23_kernel,
        grid=(2 * NBLKB,),
        in_specs=[
            pl.BlockSpec((BLKB, N), lambda i: (jax.lax.rem(i, NBLKB), 0)),
            full((N, NHID)),
            full((NHID, NHID)),
            full((NHID, NHID)),
            full((1, NHID)),
            full((1, NHID)),
            full((NCLASS, 3 * NHID)),
            full((1, NCLASS)),
            full((1, NHID)),
        ],
        out_specs=pl.BlockSpec((1, NCLASS), lambda i: (0, 0)),
        out_shape=jax.ShapeDtypeStruct((1, NCLASS), jnp.float32),
        scratch_shapes=[
            pltpu.VMEM((N, NHID), jnp.bfloat16),
            pltpu.VMEM((N, NHID), jnp.bfloat16),
            pltpu.VMEM((BLKB, NHID), jnp.float32),
            pltpu.VMEM((BLKB, NHID), jnp.float32),
        ],
        compiler_params=pltpu.CompilerParams(
            dimension_semantics=("arbitrary",)),
    )(adj_bf16, x1, W2, W3, b2.reshape(1, -1), b3.reshape(1, -1), linW,
      linb.reshape(1, -1), o1)
    return out.reshape(NCLASS)


# speedup vs baseline: 1.0684x; 1.0165x over previous
import jax
import jax.numpy as jnp
from jax.experimental import pallas as pl
from jax.experimental.pallas import tpu as pltpu

N, NFEAT, NHID, NCLASS = 8192, 256, 64, 16

BLKA = 512
NBLKA = N // BLKA
BLKB = 1024
NBLKB = N // BLKB
KSPL = N // 2


def _ksplit_dot(a, h_ref):
    return (jnp.dot(a[:, :KSPL], h_ref[:KSPL, :],
                    preferred_element_type=jnp.float32)
            + jnp.dot(a[:, KSPL:], h_ref[KSPL:, :],
                      preferred_element_type=jnp.float32))


def _h1_kernel(x_ref, W1_ref, h1_ref):
    h1_ref[...] = jnp.dot(x_ref[...], W1_ref[...],
                          preferred_element_type=jnp.float32
                          ).astype(jnp.bfloat16)


def _layer1_kernel(adj_ref, h1_ref, b1_ref,
                   adjb_ref, x1_ref, o1_ref, acc_ref):
    j = pl.program_id(0)

    ab = adj_ref[...].astype(jnp.bfloat16)
    adjb_ref[...] = ab
    y = _ksplit_dot(ab, h1_ref)
    yr = jnp.maximum(y + b1_ref[...], 0.0)
    x1_ref[...] = yr.astype(jnp.bfloat16)

    @pl.when(j == 0)
    def _():
        acc_ref[...] = yr

    @pl.when(j != 0)
    def _():
        acc_ref[...] = jnp.maximum(acc_ref[...], yr)

    @pl.when(j == NBLKA - 1)
    def _():
        o1_ref[...] = jnp.max(acc_ref[...], axis=0, keepdims=True)


def _layer23_kernel(adjb_ref, x1_ref, W2_ref, W3_ref, b2_ref, b3_ref,
                    linW_ref, linb_ref, o1_ref, out_ref,
                    hcur_ref, h3_ref, acc2_ref, acc3_ref):
    i = pl.program_id(0)
    j = jax.lax.rem(i, NBLKB)
    l = jax.lax.div(i, NBLKB)

    @pl.when(i == 0)
    def _():
        hcur_ref[...] = jnp.dot(x1_ref[...], W2_ref[...],
                                preferred_element_type=jnp.float32
                                ).astype(jnp.bfloat16)

    @pl.when(i == NBLKB)
    def _():
        hcur_ref[...] = h3_ref[...]

    y = _ksplit_dot(adjb_ref[...], hcur_ref)
    y = y + jnp.where(l == 0, b2_ref[...], b3_ref[...])
    yr = jnp.maximum(y, 0.0)

    @pl.when(l == 0)
    def _():
        h3_ref[pl.ds(j * BLKB, BLKB), :] = jnp.dot(
            yr, W3_ref[...],
            preferred_element_type=jnp.float32).astype(jnp.bfloat16)

        @pl.when(j == 0)
        def _():
            acc2_ref[...] = yr

        @pl.when(j != 0)
        def _():
            acc2_ref[...] = jnp.maximum(acc2_ref[...], yr)

    @pl.when(l == 1)
    def _():
        @pl.when(j == 0)
        def _():
            acc3_ref[...] = y

        @pl.when(j != 0)
        def _():
            acc3_ref[...] = jnp.maximum(acc3_ref[...], y)

    @pl.when(i == 2 * NBLKB - 1)
    def _():
        o2 = jnp.max(acc2_ref[...], axis=0, keepdims=True)
        o3 = jnp.max(acc3_ref[...], axis=0, keepdims=True)
        logits = (jnp.sum(linW_ref[:, 0:NHID] * o1_ref[...], axis=1)
                  + jnp.sum(linW_ref[:, NHID:2 * NHID] * o2, axis=1)
                  + jnp.sum(linW_ref[:, 2 * NHID:] * o3, axis=1)
                  + linb_ref[0, :])
        z = logits - jnp.max(logits)
        out_ref[0, :] = z - jnp.log(jnp.sum(jnp.exp(z)))


def kernel(x, adj, W1, b1, W2, b2, W3, b3, linW, linb):
    full = lambda shape: pl.BlockSpec(shape, lambda i: (0, 0))

    h1 = pl.pallas_call(
        _h1_kernel,
        out_shape=jax.ShapeDtypeStruct((N, NHID), jnp.bfloat16),
    )(x, W1)

    adj_bf16, x1, o1 = pl.pallas_call(
        _layer1_kernel,
        grid=(NBLKA,),
        in_specs=[
            pl.BlockSpec((BLKA, N), lambda j: (j, 0)),
            full((N, NHID)),
            full((1, NHID)),
        ],
        out_specs=[
            pl.BlockSpec((BLKA, N), lambda j: (j, 0)),
            pl.BlockSpec((BLKA, NHID), lambda j: (j, 0)),
            pl.BlockSpec((1, NHID), lambda j: (0, 0)),
        ],
        out_shape=[
            jax.ShapeDtypeStruct((N, N), jnp.bfloat16),
            jax.ShapeDtypeStruct((N, NHID), jnp.bfloat16),
            jax.ShapeDtypeStruct((1, NHID), jnp.float32),
        ],
        scratch_shapes=[
            pltpu.VMEM((BLKA, NHID), jnp.float32),
        ],
        compiler_params=pltpu.CompilerParams(
            dimension_semantics=("arbitrary",)),
    )(adj, h1, b1.reshape(1, -1))

    out = pl.pallas_call(
        _layer23_kernel,
        grid=(2 * NBLKB,),
        in_specs=[
            pl.BlockSpec((BLKB, N), lambda i: (jax.lax.rem(i, NBLKB), 0)),
            full((N, NHID)),
            full((NHID, NHID)),
            full((NHID, NHID)),
            full((1, NHID)),
            full((1, NHID)),
            full((NCLASS, 3 * NHID)),
            full((1, NCLASS)),
            full((1, NHID)),
        ],
        out_specs=pl.BlockSpec((1, NCLASS), lambda i: (0, 0)),
        out_shape=jax.ShapeDtypeStruct((1, NCLASS), jnp.float32),
        scratch_shapes=[
            pltpu.VMEM((N, NHID), jnp.bfloat16),
            pltpu.VMEM((N, NHID), jnp.bfloat16),
            pltpu.VMEM((BLKB, NHID), jnp.float32),
            pltpu.VMEM((BLKB, NHID), jnp.float32),
        ],
        compiler_params=pltpu.CompilerParams(
            dimension_semantics=("arbitrary",)),
    )(adj_bf16, x1, W2, W3, b2.reshape(1, -1), b3.reshape(1, -1), linW,
      linb.reshape(1, -1), o1)
    return out.reshape(NCLASS)
